# Initial kernel scaffold; baseline (speedup 1.0000x reference)
import jax
import jax.numpy as jnp
from jax.experimental import pallas as pl


def kernel(x, batch, edge, W, b):
    raise NotImplementedError("write your pallas kernel here")


# speedup vs baseline: 2.8264x; 2.8264x over previous
import functools

import jax
import jax.numpy as jnp
from jax import lax
from jax.experimental import pallas as pl
from jax.experimental.pallas import tpu as pltpu
from jax.experimental.pallas import tpu_sc as plsc

M = 10000
D = 128
G = 64
L = 16
NC = 2
NS = 16
NW = NC * NS
CHUNK = 320
NV = CHUNK // L
PADM = NW * CHUNK
NEG = -1e30

_mesh = plsc.VectorSubcoreMesh(
    core_axis_name="c", subcore_axis_name="s", num_cores=NC, num_subcores=NS)
_params = pltpu.CompilerParams(needs_layout_passes=False)


def _wid_base():
    wid = lax.axis_index("c") * NS + lax.axis_index("s")
    base = wid * CHUNK
    nvalid = jnp.minimum(CHUNK, M - base)
    ngrp = (nvalid + L - 1) // L
    return wid, base, ngrp


def _k1_body(x_hbm, batch_hbm, w_hbm, imp_hbm, pmax_hbm, psum_hbm,
             xbuf, bbuf, wvbuf, impbuf, pmaxbuf, psumbuf):
    wid, base, ngrp = _wid_base()

    pltpu.sync_copy(w_hbm, wvbuf)
    pltpu.sync_copy(batch_hbm.at[pl.ds(base, CHUNK)], bbuf)
    wv = [wvbuf[pl.ds(c * L, L)] for c in range(D // L)]

    neg = jnp.full((L,), NEG, jnp.float32)
    for j in range(NV):
        impbuf[pl.ds(j * L, L)] = neg

    lanes = lax.iota(jnp.int32, L)

    def grp(g, _):
        pltpu.sync_copy(x_hbm.at[pl.ds(base + g * L, L)], xbuf)

        def row(r, impv):
            acc = xbuf[r, pl.ds(0, L)] * wv[0]
            for c in range(1, D // L):
                acc = acc + xbuf[r, pl.ds(c * L, L)] * wv[c]
            val = jnp.sum(acc)
            return jnp.where(lanes == r, val, impv)

        impv = lax.fori_loop(0, L, row, neg)
        impbuf[pl.ds(g * L, L)] = impv
        return 0

    lax.fori_loop(0, ngrp, grp, 0)

    zero = jnp.zeros((L,), jnp.float32)
    for q in range(G // L):
        pmaxbuf[pl.ds(q * L, L)] = neg
        psumbuf[pl.ds(q * L, L)] = zero

    g_lo = jnp.min(bbuf[pl.ds(0, L)])
    g_hi = jnp.max(bbuf[pl.ds((ngrp - 1) * L, L)])

    def seg(g, _):
        macc = neg
        for j in range(NV):
            bm = bbuf[pl.ds(j * L, L)] == g
            macc = jnp.maximum(macc, jnp.where(bm, impbuf[pl.ds(j * L, L)], NEG))
        mg = jnp.max(macc)
        sacc = zero
        for j in range(NV):
            bm = bbuf[pl.ds(j * L, L)] == g
            e = jnp.exp(impbuf[pl.ds(j * L, L)] - mg)
            sacc = sacc + jnp.where(bm, e, 0.0)
        sg = jnp.sum(sacc)
        q, lane = g // L, g % L
        hit = lanes == lane
        pmaxbuf[pl.ds(q * L, L)] = jnp.where(hit, mg, pmaxbuf[pl.ds(q * L, L)])
        psumbuf[pl.ds(q * L, L)] = jnp.where(hit, sg, psumbuf[pl.ds(q * L, L)])
        return 0

    lax.fori_loop(g_lo, g_hi + 1, seg, 0)

    pltpu.sync_copy(impbuf, imp_hbm.at[pl.ds(base, CHUNK)])
    pltpu.sync_copy(pmaxbuf, pmax_hbm.at[wid])
    pltpu.sync_copy(psumbuf, psum_hbm.at[wid])


def _k2_body(x_hbm, batch_hbm, imp_hbm, pmax_hbm, psum_hbm, out_hbm,
             xbuf, bbuf, impbuf, wbuf, pmaxs, psums, mbuf, sbuf):
    wid, base, ngrp = _wid_base()

    pltpu.sync_copy(pmax_hbm, pmaxs)
    pltpu.sync_copy(psum_hbm, psums)
    pltpu.sync_copy(batch_hbm.at[pl.ds(base, CHUNK)], bbuf)
    pltpu.sync_copy(imp_hbm.at[pl.ds(base, CHUNK)], impbuf)

    NQ = G // L

    def mrow(r, carry):
        return tuple(jnp.maximum(carry[q], pmaxs[r, pl.ds(q * L, L)])
                     for q in range(NQ))

    m = lax.fori_loop(0, NW, mrow,
                      tuple(jnp.full((L,), NEG, jnp.float32) for _ in range(NQ)))

    def srow(r, carry):
        return tuple(carry[q] + psums[r, pl.ds(q * L, L)]
                     * jnp.exp(pmaxs[r, pl.ds(q * L, L)] - m[q])
                     for q in range(NQ))

    s = lax.fori_loop(0, NW, srow,
                      tuple(jnp.zeros((L,), jnp.float32) for _ in range(NQ)))
    for q in range(NQ):
        mbuf[pl.ds(q * L, L)] = m[q]
        sbuf[pl.ds(q * L, L)] = s[q]

    for j in range(NV):
        bv = bbuf[pl.ds(j * L, L)]
        mv = plsc.load_gather(mbuf, [bv])
        sv = plsc.load_gather(sbuf, [bv])
        wbuf[pl.ds(j * L, L)] = jnp.exp(impbuf[pl.ds(j * L, L)] - mv) / sv

    def grp(g, _):
        pltpu.sync_copy(x_hbm.at[pl.ds(base + g * L, L)], xbuf)

        def row(r, _):
            wr = plsc.load_gather(wbuf, [jnp.full((L,), g * L, jnp.int32) + r])
            for c in range(D // L):
                xbuf[r, pl.ds(c * L, L)] = xbuf[r, pl.ds(c * L, L)] * wr
            return 0

        lax.fori_loop(0, L, row, 0)
        pltpu.sync_copy(xbuf, out_hbm.at[pl.ds(base + g * L, L)])
        return 0

    lax.fori_loop(0, ngrp, grp, 0)


_k1 = pl.kernel(
    _k1_body,
    out_type=(
        jax.ShapeDtypeStruct((PADM,), jnp.float32),
        jax.ShapeDtypeStruct((NW, G), jnp.float32),
        jax.ShapeDtypeStruct((NW, G), jnp.float32),
    ),
    mesh=_mesh,
    compiler_params=_params,
    scratch_types=[
        pltpu.VMEM((L, D), jnp.float32),
        pltpu.VMEM((CHUNK,), jnp.int32),
        pltpu.VMEM((D,), jnp.float32),
        pltpu.VMEM((CHUNK,), jnp.float32),
        pltpu.VMEM((G,), jnp.float32),
        pltpu.VMEM((G,), jnp.float32),
    ],
)

_k2 = pl.kernel(
    _k2_body,
    out_type=jax.ShapeDtypeStruct((M, D), jnp.float32),
    mesh=_mesh,
    compiler_params=_params,
    scratch_types=[
        pltpu.VMEM((L, D), jnp.float32),
        pltpu.VMEM((CHUNK,), jnp.int32),
        pltpu.VMEM((CHUNK,), jnp.float32),
        pltpu.VMEM((CHUNK,), jnp.float32),
        pltpu.VMEM((NW, G), jnp.float32),
        pltpu.VMEM((NW, G), jnp.float32),
        pltpu.VMEM((G,), jnp.float32),
        pltpu.VMEM((G,), jnp.float32),
    ],
)


def kernel(x, batch, edge, W, b):
    del edge, b
    batch_pad = jnp.pad(batch, (0, PADM - M), constant_values=G - 1)
    wflat = W.reshape(D)
    imp, pmax, psum = _k1(x, batch_pad, wflat)
    return _k2(x, batch_pad, imp, pmax, psum)
